# baseline (speedup 1.0000x reference)
import jax
import jax.numpy as jnp
from jax import lax
from jax.experimental import pallas as pl
from jax.experimental.pallas import tpu as pltpu
from jax.experimental.pallas import tpu_sc as plsc

LENGTH = 100000
FEAT_DIM = 128
BATCH = 16384

NC = 2
NS = 16
NW = NC * NS
B_PER_W = BATCH // NW
SUB = 128
NSUB = B_PER_W // SUB
LANE = 16

_MESH = plsc.VectorSubcoreMesh(core_axis_name="c", subcore_axis_name="s")
_SC_PARAMS = pltpu.CompilerParams(needs_layout_passes=False)


def _vrsqrt(sv):
    sc = jnp.maximum(sv, 1e-24)
    i = lax.bitcast_convert_type(sc, jnp.int32)
    i = 0x5F3759DF - lax.shift_right_logical(i, 1)
    y = lax.bitcast_convert_type(i, jnp.float32)
    for _ in range(3):
        y = y * (1.5 - 0.5 * sc * y * y)
    return y


def _row_normalize(ob, fb, n_rows):
    @plsc.parallel_loop(0, n_rows, unroll=2)
    def _rows(r):
        u = [ob[r, pl.ds(16 * k, 16)] + fb[r, pl.ds(16 * k, 16)]
             for k in range(8)]
        ss = u[0] * u[0]
        for k in range(1, 8):
            ss = ss + u[k] * u[k]
        y = _vrsqrt(jnp.broadcast_to(jnp.sum(ss), (LANE,)))
        for k in range(8):
            fb[r, pl.ds(16 * k, 16)] = u[k] * y


def _winners_body(ind_hbm, j_hbm, w_ref, idxb):
    cid = lax.axis_index("c")
    sid = lax.axis_index("s")
    lanes = lax.iota(jnp.int32, LANE)

    @pl.when((sid == 0) & (cid == 0))
    def _():
        pltpu.sync_copy(ind_hbm, idxb)

        UN = 8

        def c_loop(cg, c2):
            for u in range(UN):
                off = cg * (LANE * UN) + u * LANE
                v = idxb[pl.ds(off, LANE)]
                plsc.store_scatter(w_ref, [v], off + lanes)
            return c2

        lax.fori_loop(0, BATCH // (LANE * UN), c_loop, 0)

        @plsc.parallel_loop(0, BATCH // LANE, unroll=4)
        def _p2(ci):
            v = idxb[pl.ds(ci * LANE, LANE)]
            idxb[pl.ds(ci * LANE, LANE)] = plsc.load_gather(w_ref, [v])

        pltpu.sync_copy(idxb, j_hbm)


_sc_winners = pl.kernel(
    _winners_body,
    out_type=jax.ShapeDtypeStruct((BATCH,), jnp.int32),
    mesh=_MESH,
    compiler_params=_SC_PARAMS,
    scratch_types=[
        pltpu.VMEM((LENGTH,), jnp.int32),
        pltpu.VMEM((BATCH,), jnp.int32),
    ],
)


def _apply_body(bank_hbm, ind_hbm, fnorm_hbm, j_hbm, out_hbm,
                ibuf, jbuf, ivs,
                oldv0, oldv1, fnv0, fnv1,
                gsem0, gsem1, ssem0, ssem1):
    cid = lax.axis_index("c")
    sid = lax.axis_index("s")
    wid = sid * NC + cid
    base = wid * B_PER_W
    pltpu.sync_copy(ind_hbm.at[pl.ds(base, B_PER_W)], ibuf)
    pltpu.sync_copy(j_hbm.at[pl.ds(base, B_PER_W)], jbuf)

    for s in range(NSUB):
        for k in range(SUB // LANE):
            ivs[s, pl.ds(k * LANE, LANE)] = ibuf[pl.ds(s * SUB + k * LANE, LANE)]

    oldv = [oldv0, oldv1]
    fnv = [fnv0, fnv1]
    gsem = [gsem0, gsem1]
    ssem = [ssem0, ssem1]

    def start_gathers(s, b):
        c1 = pltpu.async_copy(
            bank_hbm.at[ibuf.at[pl.ds(s * SUB, SUB)]], oldv[b], gsem[b])
        c2 = pltpu.async_copy(
            fnorm_hbm.at[jbuf.at[pl.ds(s * SUB, SUB)]], fnv[b], gsem[b])
        return (c1, c2)

    gd = [None, None]
    sd = [None, None]
    gd[0] = start_gathers(0, 0)
    for s in range(NSUB):
        b = s & 1
        nb = 1 - b
        if s + 1 < NSUB:
            if sd[nb] is not None:
                sd[nb].wait()
                sd[nb] = None
            gd[nb] = start_gathers(s + 1, nb)
        gd[b][0].wait()
        gd[b][1].wait()

        _row_normalize(oldv[b], fnv[b], SUB)
        sd[b] = pltpu.async_copy(fnv[b], out_hbm.at[ivs.at[s]], ssem[b])
    for b in range(2):
        if sd[b] is not None:
            sd[b].wait()


_sc_apply = pl.kernel(
    _apply_body,
    out_type=(),
    mesh=_MESH,
    compiler_params=_SC_PARAMS,
    scratch_types=[
        pltpu.VMEM((B_PER_W,), jnp.int32),
        pltpu.VMEM((B_PER_W,), jnp.int32),
        pltpu.VMEM((NSUB, SUB), jnp.int32),
        pltpu.VMEM((SUB, FEAT_DIM), jnp.float32),
        pltpu.VMEM((SUB, FEAT_DIM), jnp.float32),
        pltpu.VMEM((SUB, FEAT_DIM), jnp.float32),
        pltpu.VMEM((SUB, FEAT_DIM), jnp.float32),
        pltpu.SemaphoreType.DMA,
        pltpu.SemaphoreType.DMA,
        pltpu.SemaphoreType.DMA,
        pltpu.SemaphoreType.DMA,
    ],
)


def _fnorm_body(x_ref, o_ref):
    x = x_ref[...]
    ss = jnp.sum(x * x, axis=1, keepdims=True)
    o_ref[...] = x * lax.rsqrt(jnp.maximum(ss, 1e-24))


_FN_BLOCK = 2048

_fnorm_call = pl.pallas_call(
    _fnorm_body,
    out_shape=jax.ShapeDtypeStruct((BATCH, FEAT_DIM), jnp.float32),
    grid=(BATCH // _FN_BLOCK,),
    in_specs=[pl.BlockSpec((_FN_BLOCK, FEAT_DIM), lambda i: (i, 0))],
    out_specs=pl.BlockSpec((_FN_BLOCK, FEAT_DIM), lambda i: (i, 0)),
)


def kernel(feature_bank, ind, feature):
    ind32 = ind.astype(jnp.int32)
    j = _sc_winners(ind32)
    out_ref = jax.new_ref(feature_bank)
    fnorm = _fnorm_call(feature)
    _sc_apply(feature_bank, ind32, fnorm, j, out_ref)
    return out_ref[...]
